# baseline (speedup 1.0000x reference)
import jax
import jax.numpy as jnp
from jax.experimental import pallas as pl
from jax.experimental.pallas import tpu as pltpu


def _tc_body(dp_ref, x_ref, out_ref):
    blk = dp_ref[...]
    nb, k3, a, d = blk.shape
    z = blk.reshape(nb, k3 * a, d) * x_ref[:, :, :]
    y = jnp.sum(z, axis=2)
    out_ref[...] = jnp.sum(y * y, axis=1).reshape(nb, 1, 1)


def kernel(x, der_desc_wrt_coord):
    B, A, D, K = der_desc_wrt_coord.shape
    dp = jnp.transpose(der_desc_wrt_coord, (0, 3, 1, 2))
    x3 = x.reshape(B, 1, D)

    NB = 16
    grid = (B // NB,)
    out = pl.pallas_call(
        _tc_body,
        grid=grid,
        in_specs=[
            pl.BlockSpec((NB, K, A, D), lambda b: (b, 0, 0, 0)),
            pl.BlockSpec((NB, 1, D), lambda b: (b, 0, 0)),
        ],
        out_specs=pl.BlockSpec((NB, 1, 1), lambda b: (b, 0, 0)),
        out_shape=jax.ShapeDtypeStruct((B, 1, 1), jnp.float32),
        compiler_params=pltpu.CompilerParams(
            dimension_semantics=("arbitrary",),
        ),
    )(dp, x3)
    return out.reshape(B)


# speedup vs baseline: 1.0265x; 1.0265x over previous
import jax
import jax.numpy as jnp
from jax.experimental import pallas as pl
from jax.experimental.pallas import tpu as pltpu


def _sumsq_rows(blk, xb):
    nb, k3, a, d = blk.shape
    z = blk.reshape(nb, k3 * a, d) * xb
    y = jnp.sum(z, axis=2)
    return jnp.sum(y * y, axis=1)


def _tc_body(dpa_ref, dpb_ref, x_ref, out_ref):
    xb = x_ref[:, :, :]
    val = _sumsq_rows(dpa_ref[...], xb) + _sumsq_rows(dpb_ref[...], xb)
    out_ref[...] = val.reshape(-1, 1, 1)


def kernel(x, der_desc_wrt_coord):
    B, A, D, K = der_desc_wrt_coord.shape
    dp = jnp.transpose(der_desc_wrt_coord, (0, 3, 1, 2))
    x3 = x.reshape(B, 1, D)

    NB = 8
    A2 = A // 2
    grid = (B // NB,)
    out = pl.pallas_call(
        _tc_body,
        grid=grid,
        in_specs=[
            pl.BlockSpec((NB, K, A2, D), lambda b: (b, 0, 0, 0)),
            pl.BlockSpec((NB, K, A2, D), lambda b: (b, 0, 1, 0)),
            pl.BlockSpec((NB, 1, D), lambda b: (b, 0, 0)),
        ],
        out_specs=pl.BlockSpec((NB, 1, 1), lambda b: (b, 0, 0)),
        out_shape=jax.ShapeDtypeStruct((B, 1, 1), jnp.float32),
        compiler_params=pltpu.CompilerParams(
            dimension_semantics=("arbitrary",),
        ),
    )(dp, dp, x3)
    return out.reshape(B)
